# Initial kernel scaffold; baseline (speedup 1.0000x reference)
import jax
import jax.numpy as jnp
from jax.experimental import pallas as pl


def kernel(x, edge_index, W_shared, b_shared, W_mu, b_mu, W_logvar, b_logvar):
    raise NotImplementedError("write your pallas kernel here")


# speedup vs baseline: 21.9967x; 21.9967x over previous
import functools

import jax
import jax.numpy as jnp
from jax import lax
from jax.experimental import pallas as pl
from jax.experimental.pallas import tpu as pltpu
from jax.experimental.pallas import tpu_sc as plsc

N = 10000
NP = 10240
E = 320000
D_IN = 128
D_H = 128
DP = 16
K = 128
NCHUNK = E // K
NC = 2
NS = 16
NW = NC * NS
RPT = NP // NS

_MESH = plsc.VectorSubcoreMesh(
    core_axis_name="c", subcore_axis_name="s", num_cores=NC, num_subcores=NS
)


def _wid():
    return lax.axis_index("s") * NC + lax.axis_index("c")


def _chunk_range(wid):
    lo = (wid * NCHUNK) // NW
    hi = ((wid + 1) * NCHUNK) // NW
    return lo, hi


def _deg_body(dst_hbm, ones_hbm, zeros_hbm, deg_out, ones_v, idx_v, deg_sh):
    cid = lax.axis_index("c")
    sid = lax.axis_index("s")
    base = sid * RPT
    pltpu.sync_copy(ones_hbm, ones_v)
    pltpu.sync_copy(zeros_hbm.at[pl.ds(base, RPT)], deg_sh.at[pl.ds(base, RPT)])
    plsc.subcore_barrier()
    lo, hi = _chunk_range(_wid())

    def body(c, carry):
        pltpu.sync_copy(dst_hbm.at[c], idx_v)
        pltpu.sync_copy(ones_v, deg_sh.at[idx_v], add=True)
        return carry

    lax.fori_loop(lo, hi, body, 0)
    plsc.subcore_barrier()
    pltpu.sync_copy(deg_sh.at[pl.ds(base, RPT)], deg_out.at[cid, pl.ds(base, RPT)])


_deg_call = pl.kernel(
    _deg_body,
    out_type=jax.ShapeDtypeStruct((NC, NP, DP), jnp.float32),
    mesh=_MESH,
    scratch_types=[
        pltpu.VMEM((K, DP), jnp.float32),
        pltpu.VMEM((K,), jnp.int32),
        pltpu.VMEM_SHARED((NP, DP), jnp.float32),
    ],
    compiler_params=pltpu.CompilerParams(use_tc_tiling_on_sc=False),
)


def _prop_body(src_hbm, dst_hbm, y_hbm, zeros_hbm, acc_out,
               src_v, dst_v, rows_v, acc_sh, sem):
    cid = lax.axis_index("c")
    sid = lax.axis_index("s")
    base = sid * RPT
    pltpu.sync_copy(zeros_hbm.at[pl.ds(base, RPT)], acc_sh.at[pl.ds(base, RPT)])
    plsc.subcore_barrier()
    lo, hi = _chunk_range(_wid())

    def body(c, carry):
        pltpu.sync_copy(src_hbm.at[c], src_v)
        pltpu.sync_copy(dst_hbm.at[c], dst_v)
        pltpu.async_copy(y_hbm.at[src_v], rows_v, sem).wait()
        pltpu.sync_copy(rows_v, acc_sh.at[dst_v], add=True)
        return carry

    lax.fori_loop(lo, hi, body, 0)
    plsc.subcore_barrier()
    pltpu.sync_copy(acc_sh.at[pl.ds(base, RPT)], acc_out.at[cid, pl.ds(base, RPT)])


def _make_prop(d, tc_tiling):
    return pl.kernel(
        _prop_body,
        out_type=jax.ShapeDtypeStruct((NC, NP, d), jnp.float32),
        mesh=_MESH,
        scratch_types=[
            pltpu.VMEM((K,), jnp.int32),
            pltpu.VMEM((K,), jnp.int32),
            pltpu.VMEM((K, d), jnp.float32),
            pltpu.VMEM_SHARED((NP, d), jnp.float32),
            pltpu.SemaphoreType.DMA,
        ],
        compiler_params=pltpu.CompilerParams(use_tc_tiling_on_sc=tc_tiling),
    )


_prop_wide = _make_prop(D_H, True)
_prop_narrow = _make_prop(DP, False)


_BLK = 1024
_GRID = NP // _BLK


def _dinv(deg_ref):
    deg = deg_ref[0, :, 0:1] + deg_ref[1, :, 0:1] + 1.0
    return lax.rsqrt(deg)


def _tc1_body(x_ref, w_ref, deg_ref, y_ref):
    xw = jnp.dot(x_ref[...], w_ref[...], preferred_element_type=jnp.float32)
    y_ref[...] = _dinv(deg_ref) * xw


def _tc2_body(acc_ref, y1_ref, deg_ref, wcat_ref, b_ref, y2_ref):
    dinv = _dinv(deg_ref)
    s = acc_ref[0] + acc_ref[1] + y1_ref[...]
    h = jnp.maximum(dinv * s + b_ref[...], 0.0)
    xw2 = jnp.dot(h, wcat_ref[...], preferred_element_type=jnp.float32)
    y2_ref[...] = dinv * xw2


def _tc3_body(acc2_ref, y2_ref, deg_ref, bcat_ref, out_ref):
    dinv = _dinv(deg_ref)
    s = acc2_ref[0] + acc2_ref[1] + y2_ref[...]
    out_ref[...] = dinv * s + bcat_ref[...]


def _deg_spec():
    return pl.BlockSpec((NC, _BLK, DP), lambda i: (0, i, 0))


_tc1_call = pl.pallas_call(
    _tc1_body,
    grid=(_GRID,),
    in_specs=[
        pl.BlockSpec((_BLK, D_IN), lambda i: (i, 0)),
        pl.BlockSpec((D_IN, D_H), lambda i: (0, 0)),
        _deg_spec(),
    ],
    out_specs=pl.BlockSpec((_BLK, D_H), lambda i: (i, 0)),
    out_shape=jax.ShapeDtypeStruct((NP, D_H), jnp.float32),
)

_tc2_call = pl.pallas_call(
    _tc2_body,
    grid=(_GRID,),
    in_specs=[
        pl.BlockSpec((NC, _BLK, D_H), lambda i: (0, i, 0)),
        pl.BlockSpec((_BLK, D_H), lambda i: (i, 0)),
        _deg_spec(),
        pl.BlockSpec((D_H, DP), lambda i: (0, 0)),
        pl.BlockSpec((1, D_H), lambda i: (0, 0)),
    ],
    out_specs=pl.BlockSpec((_BLK, DP), lambda i: (i, 0)),
    out_shape=jax.ShapeDtypeStruct((NP, DP), jnp.float32),
)

_tc3_call = pl.pallas_call(
    _tc3_body,
    grid=(_GRID,),
    in_specs=[
        pl.BlockSpec((NC, _BLK, DP), lambda i: (0, i, 0)),
        pl.BlockSpec((_BLK, DP), lambda i: (i, 0)),
        _deg_spec(),
        pl.BlockSpec((1, DP), lambda i: (0, 0)),
    ],
    out_specs=pl.BlockSpec((_BLK, DP), lambda i: (i, 0)),
    out_shape=jax.ShapeDtypeStruct((NP, DP), jnp.float32),
)


@jax.jit
def kernel(x, edge_index, W_shared, b_shared, W_mu, b_mu, W_logvar, b_logvar):
    src2d = edge_index[0].reshape(NCHUNK, K)
    dst2d = edge_index[1].reshape(NCHUNK, K)
    xp = jnp.zeros((NP, D_IN), jnp.float32).at[:N].set(x)
    ones_kp = jnp.ones((K, DP), jnp.float32)
    zeros_np = jnp.zeros((NP, DP), jnp.float32)
    zeros_nh = jnp.zeros((NP, D_H), jnp.float32)

    deg_parts = _deg_call(dst2d, ones_kp, zeros_np)
    y1 = _tc1_call(xp, W_shared, deg_parts)
    acc1 = _prop_wide(src2d, dst2d, y1, zeros_nh)

    wcat = jnp.concatenate(
        [W_mu, W_logvar, jnp.zeros((D_H, DP - 4), jnp.float32)], axis=1)
    bcat = jnp.concatenate(
        [b_mu, b_logvar, jnp.zeros((DP - 4,), jnp.float32)]).reshape(1, DP)

    y2 = _tc2_call(acc1, y1, deg_parts, wcat, b_shared.reshape(1, D_H))
    acc2 = _prop_narrow(src2d, dst2d, y2, zeros_np)
    out2 = _tc3_call(acc2, y2, deg_parts, bcat)
    return out2[:N, 0:2], out2[:N, 2:4]
